# baseline (speedup 1.0000x reference)
import jax
import jax.numpy as jnp
from jax.experimental import pallas as pl
from jax.experimental.pallas import tpu as pltpu

_SHIFT = 8192
_LANE = 128

_DEPTH = 24
_AHEAD = 12


def _make_tc_copy(s, b, c, in_len, out_len, dtype):
    total_tiles = out_len // _LANE
    nsplit = 4
    base, rem = divmod(total_tiles, nsplit)
    lens = [base + (1 if i < rem else 0) for i in range(nsplit)]
    chunks, acc = [], 0
    for ln in lens:
        chunks.append((acc, ln))
        acc += ln
    tasks = [(si, bi, off, ln) for si in range(s) for bi in range(b)
             for off, ln in chunks]
    ntask = len(tasks)
    max_tiles = max(ln for _, ln in chunks)

    def body(in_ref, out_ref, *rest):
        bufs, (rsem, wsem) = rest[:_DEPTH], rest[_DEPTH:]

        def read_copy(t):
            si, bi, off, ln = tasks[t]
            slot = t % _DEPTH
            return pltpu.make_async_copy(
                in_ref.at[si, bi, :, pl.ds(off * _LANE, ln * _LANE)],
                bufs[slot].at[:, pl.ds(0, ln * _LANE)],
                rsem.at[slot],
            )

        def write_copy(t):
            si, bi, off, ln = tasks[t]
            slot = t % _DEPTH
            return pltpu.make_async_copy(
                bufs[slot].at[:, pl.ds(0, ln * _LANE)],
                out_ref.at[si, bi, :, pl.ds(off * _LANE, ln * _LANE)],
                wsem.at[slot],
            )

        for t in range(min(_AHEAD, ntask)):
            read_copy(t).start()
        for t in range(ntask):
            nt = t + _AHEAD
            if nt < ntask:
                if nt >= _DEPTH:
                    write_copy(nt - _DEPTH).wait()
                read_copy(nt).start()
            read_copy(t).wait()
            write_copy(t).start()
        for t in range(max(0, ntask - _DEPTH), ntask):
            write_copy(t).wait()

    return pl.pallas_call(
        body,
        in_specs=[pl.BlockSpec(memory_space=pl.ANY)],
        out_specs=pl.BlockSpec(memory_space=pl.ANY),
        out_shape=jax.ShapeDtypeStruct((s, b, c, out_len), dtype),
        scratch_shapes=[pltpu.VMEM((c, max_tiles * _LANE), dtype)] * _DEPTH + [
            pltpu.SemaphoreType.DMA((_DEPTH,)),
            pltpu.SemaphoreType.DMA((_DEPTH,)),
        ],
    )


def kernel(wav):
    s, b, c, length = wav.shape
    out_len = length - _SHIFT
    return _make_tc_copy(s, b, c, length, out_len, wav.dtype)(wav)


# speedup vs baseline: 1.0239x; 1.0239x over previous
import jax
import jax.numpy as jnp
from jax.experimental import pallas as pl
from jax.experimental.pallas import tpu as pltpu

_SHIFT = 8192
_LANE = 128

_DEPTH = 20
_AHEAD = 10


def _make_tc_copy(s, b, c, in_len, out_len, dtype):
    total_tiles = out_len // _LANE
    nsplit = 2
    base, rem = divmod(total_tiles, nsplit)
    lens = [base + (1 if i < rem else 0) for i in range(nsplit)]
    chunks, acc = [], 0
    for ln in lens:
        chunks.append((acc, ln))
        acc += ln
    tasks = [(si, bi, off, ln) for si in range(s) for bi in range(b)
             for off, ln in chunks]
    ntask = len(tasks)
    max_tiles = max(ln for _, ln in chunks)

    def body(in_ref, out_ref, *rest):
        bufs, (rsem, wsem) = rest[:_DEPTH], rest[_DEPTH:]

        def read_copy(t):
            si, bi, off, ln = tasks[t]
            slot = t % _DEPTH
            return pltpu.make_async_copy(
                in_ref.at[si, bi, :, pl.ds(off * _LANE, ln * _LANE)],
                bufs[slot].at[:, pl.ds(0, ln * _LANE)],
                rsem.at[slot],
            )

        def write_copy(t):
            si, bi, off, ln = tasks[t]
            slot = t % _DEPTH
            return pltpu.make_async_copy(
                bufs[slot].at[:, pl.ds(0, ln * _LANE)],
                out_ref.at[si, bi, :, pl.ds(off * _LANE, ln * _LANE)],
                wsem.at[slot],
            )

        for t in range(min(_AHEAD, ntask)):
            read_copy(t).start()
        for t in range(ntask):
            nt = t + _AHEAD
            if nt < ntask:
                if nt >= _DEPTH:
                    write_copy(nt - _DEPTH).wait()
                read_copy(nt).start()
            read_copy(t).wait()
            write_copy(t).start()
        for t in range(max(0, ntask - _DEPTH), ntask):
            write_copy(t).wait()

    return pl.pallas_call(
        body,
        in_specs=[pl.BlockSpec(memory_space=pl.ANY)],
        out_specs=pl.BlockSpec(memory_space=pl.ANY),
        out_shape=jax.ShapeDtypeStruct((s, b, c, out_len), dtype),
        scratch_shapes=[pltpu.VMEM((c, max_tiles * _LANE), dtype)] * _DEPTH + [
            pltpu.SemaphoreType.DMA((_DEPTH,)),
            pltpu.SemaphoreType.DMA((_DEPTH,)),
        ],
    )


def kernel(wav):
    s, b, c, length = wav.shape
    out_len = length - _SHIFT
    return _make_tc_copy(s, b, c, length, out_len, wav.dtype)(wav)


# speedup vs baseline: 1.0337x; 1.0096x over previous
import jax
import jax.numpy as jnp
from jax.experimental import pallas as pl
from jax.experimental.pallas import tpu as pltpu

_SHIFT = 8192
_LANE = 128

_DEPTH = 8
_AHEAD = 4


def _make_tc_copy(s, b, c, in_len, out_len, dtype):
    total_tiles = out_len // _LANE
    nsplit = 1
    base, rem = divmod(total_tiles, nsplit)
    lens = [base + (1 if i < rem else 0) for i in range(nsplit)]
    chunks, acc = [], 0
    for ln in lens:
        chunks.append((acc, ln))
        acc += ln
    tasks = [(si, bi, off, ln) for si in range(s) for bi in range(b)
             for off, ln in chunks]
    ntask = len(tasks)
    max_tiles = max(ln for _, ln in chunks)

    def body(in_ref, out_ref, *rest):
        bufs, (rsem, wsem) = rest[:_DEPTH], rest[_DEPTH:]

        def read_copy(t):
            si, bi, off, ln = tasks[t]
            slot = t % _DEPTH
            return pltpu.make_async_copy(
                in_ref.at[si, bi, :, pl.ds(off * _LANE, ln * _LANE)],
                bufs[slot].at[:, pl.ds(0, ln * _LANE)],
                rsem.at[slot],
            )

        def write_copy(t):
            si, bi, off, ln = tasks[t]
            slot = t % _DEPTH
            return pltpu.make_async_copy(
                bufs[slot].at[:, pl.ds(0, ln * _LANE)],
                out_ref.at[si, bi, :, pl.ds(off * _LANE, ln * _LANE)],
                wsem.at[slot],
            )

        for t in range(min(_AHEAD, ntask)):
            read_copy(t).start()
        for t in range(ntask):
            nt = t + _AHEAD
            if nt < ntask:
                if nt >= _DEPTH:
                    write_copy(nt - _DEPTH).wait()
                read_copy(nt).start()
            read_copy(t).wait()
            write_copy(t).start()
        for t in range(max(0, ntask - _DEPTH), ntask):
            write_copy(t).wait()

    return pl.pallas_call(
        body,
        in_specs=[pl.BlockSpec(memory_space=pl.ANY)],
        out_specs=pl.BlockSpec(memory_space=pl.ANY),
        out_shape=jax.ShapeDtypeStruct((s, b, c, out_len), dtype),
        scratch_shapes=[pltpu.VMEM((c, max_tiles * _LANE), dtype)] * _DEPTH + [
            pltpu.SemaphoreType.DMA((_DEPTH,)),
            pltpu.SemaphoreType.DMA((_DEPTH,)),
        ],
    )


def kernel(wav):
    s, b, c, length = wav.shape
    out_len = length - _SHIFT
    return _make_tc_copy(s, b, c, length, out_len, wav.dtype)(wav)
